# baseline (speedup 1.0000x reference)
import functools

import jax
import jax.numpy as jnp
from jax import lax
from jax.experimental import pallas as pl
from jax.experimental.pallas import tpu as pltpu
from jax.experimental.pallas import tpu_sc as plsc

_CHUNK = 128
_NSLICE = 4


def _proj_nodes_body(nf_ref, w_ref, out_ref):
    out_ref[...] = jnp.dot(nf_ref[...], w_ref[...],
                           preferred_element_type=jnp.float32)


def _proj_nodes(nf, w12):
    n, d = nf.shape
    return pl.pallas_call(
        _proj_nodes_body,
        grid=(2,),
        in_specs=[
            pl.BlockSpec((n, d), lambda i: (0, 0)),
            pl.BlockSpec((d, d), lambda i: (i, 0)),
        ],
        out_specs=pl.BlockSpec((n, d), lambda i: (i, 0)),
        out_shape=jax.ShapeDtypeStruct((2 * n, d), jnp.float32),
    )(nf, w12)


def _proj_edges_body(ef_ref, w_ref, b_ref, out_ref):
    out_ref[...] = jnp.dot(ef_ref[...], w_ref[...],
                           preferred_element_type=jnp.float32) + b_ref[...]


def _proj_edges(ef, w, b, block_e, blk0, nblk):
    d = ef.shape[1]
    return pl.pallas_call(
        _proj_edges_body,
        grid=(nblk,),
        in_specs=[
            pl.BlockSpec((block_e, d), lambda i: (blk0 + i, 0)),
            pl.BlockSpec((d, d), lambda i: (0, 0)),
            pl.BlockSpec((1, d), lambda i: (0, 0)),
        ],
        out_specs=pl.BlockSpec((block_e, d), lambda i: (i, 0)),
        out_shape=jax.ShapeDtypeStruct((nblk * block_e, d), jnp.float32),
    )(ef, w, b)


def _edge_mlp2_body(of_ref, hpre_ref, ef_ref, w_ref, b_ref, g_ref, beta_ref,
                    ne_ref, out_ref):
    del of_ref
    h = jnp.maximum(hpre_ref[...], 0.0)
    y = jnp.dot(h, w_ref[...], preferred_element_type=jnp.float32) + b_ref[...]
    y = jnp.maximum(y, 0.0)
    mu = jnp.mean(y, axis=-1, keepdims=True)
    var = jnp.mean(jnp.square(y - mu), axis=-1, keepdims=True)
    ne = (y - mu) * jax.lax.rsqrt(var + 1e-5) * g_ref[...] + beta_ref[...]
    ne_ref[...] = ne
    out_ref[...] = ne + ef_ref[...]


def _edge_mlp2(of, hpre, ef, w, b, g, beta, block_e, blk0, nblk):
    e, d = ef.shape
    return pl.pallas_call(
        _edge_mlp2_body,
        grid=(nblk,),
        in_specs=[
            pl.BlockSpec(memory_space=pl.ANY),
            pl.BlockSpec((block_e, d), lambda i: (i, 0)),
            pl.BlockSpec((block_e, d), lambda i: (blk0 + i, 0)),
            pl.BlockSpec((d, d), lambda i: (0, 0)),
            pl.BlockSpec((1, d), lambda i: (0, 0)),
            pl.BlockSpec((1, d), lambda i: (0, 0)),
            pl.BlockSpec((1, d), lambda i: (0, 0)),
        ],
        out_specs=[
            pl.BlockSpec((block_e, d), lambda i: (i, 0)),
            pl.BlockSpec((block_e, d), lambda i: (blk0 + i, 0)),
        ],
        out_shape=[
            jax.ShapeDtypeStruct((nblk * block_e, d), jnp.float32),
            jax.ShapeDtypeStruct((e, d), jnp.float32),
        ],
        input_output_aliases={0: 1},
    )(of, hpre, ef, w, b, g, beta)


def _node_mlp_body(naps, nf_ref, *rest):
    ap_refs = rest[:naps]
    (w1a_ref, w1b_ref, b1_ref, w2_ref, b2_ref, g_ref, beta_ref,
     out_ref) = rest[naps:]
    nf = nf_ref[...]
    aggr = ap_refs[0][0] + ap_refs[0][1]
    for ap in ap_refs[1:]:
        aggr = aggr + ap[0] + ap[1]
    y = (jnp.dot(nf, w1a_ref[...], preferred_element_type=jnp.float32)
         + jnp.dot(aggr, w1b_ref[...], preferred_element_type=jnp.float32)
         + b1_ref[...])
    y = jnp.maximum(y, 0.0)
    y = jnp.dot(y, w2_ref[...], preferred_element_type=jnp.float32) + b2_ref[...]
    y = jnp.maximum(y, 0.0)
    mu = jnp.mean(y, axis=-1, keepdims=True)
    var = jnp.mean(jnp.square(y - mu), axis=-1, keepdims=True)
    out_ref[...] = ((y - mu) * jax.lax.rsqrt(var + 1e-5) * g_ref[...]
                    + beta_ref[...] + nf)


def _node_mlp(nf, aps, w1a, w1b, b1, w2, b2, g, beta, block_n):
    n, d = nf.shape
    nc = aps[0].shape[0]
    return pl.pallas_call(
        functools.partial(_node_mlp_body, len(aps)),
        grid=(n // block_n,),
        in_specs=[pl.BlockSpec((block_n, d), lambda i: (i, 0))]
        + [pl.BlockSpec((nc, block_n, d), lambda i: (0, i, 0))
           for _ in aps]
        + [
            pl.BlockSpec((d, d), lambda i: (0, 0)),
            pl.BlockSpec((d, d), lambda i: (0, 0)),
            pl.BlockSpec((1, d), lambda i: (0, 0)),
            pl.BlockSpec((d, d), lambda i: (0, 0)),
            pl.BlockSpec((1, d), lambda i: (0, 0)),
            pl.BlockSpec((1, d), lambda i: (0, 0)),
            pl.BlockSpec((1, d), lambda i: (0, 0)),
        ],
        out_specs=pl.BlockSpec((block_n, d), lambda i: (i, 0)),
        out_shape=jax.ShapeDtypeStruct((n, d), jnp.float32),
    )(nf, *aps, w1a, w1b, b1, w2, b2, g, beta)


def _sc_gather(table, eproj, gidx, chunk0, nchunk):
    es, d = eproj.shape
    info = plsc.get_sparse_core_info()
    nc, ns = info.num_cores, info.num_subcores
    nw = nc * ns
    full, rem = nchunk // nw, nchunk % nw
    mesh = plsc.VectorSubcoreMesh(core_axis_name="c", subcore_axis_name="s")

    @functools.partial(
        pl.kernel,
        out_type=jax.ShapeDtypeStruct((es, d), jnp.float32),
        mesh=mesh,
        scratch_types=[
            pltpu.VMEM((2, 2, _CHUNK), jnp.int32),
            pltpu.VMEM((2, _CHUNK, d), jnp.float32),
            pltpu.VMEM((2, _CHUNK, d), jnp.float32),
            pltpu.VMEM((2, _CHUNK, d), jnp.float32),
            pltpu.SemaphoreType.DMA,
            pltpu.SemaphoreType.DMA,
            pltpu.SemaphoreType.DMA,
            pltpu.SemaphoreType.DMA,
        ],
    )
    def k(table_hbm, eproj_hbm, gidx_hbm, out_hbm,
          idx_v, buf_a, buf_b, buf_e, sem_a, sem_b, sem_e, sem_o):
        cid0 = lax.axis_index("s") * nc + lax.axis_index("c")
        cnt = full + jnp.where(cid0 < rem, 1, 0) if rem else full

        def cidf(i):
            return i * nw + cid0

        def issue(i, r):
            pltpu.sync_copy(gidx_hbm.at[chunk0 + cidf(i)], idx_v.at[r])
            pltpu.async_copy(table_hbm.at[idx_v.at[r, 0]], buf_a.at[r], sem_a)
            pltpu.async_copy(table_hbm.at[idx_v.at[r, 1]], buf_b.at[r], sem_b)
            pltpu.async_copy(
                eproj_hbm.at[pl.ds(cidf(i) * _CHUNK, _CHUNK)],
                buf_e.at[r], sem_e)

        def drain_out():
            pltpu.make_async_copy(
                buf_e.at[0], out_hbm.at[pl.ds(0, _CHUNK)], sem_o).wait()

        issue(0, 0)

        def body(i, _):
            r = lax.rem(i, 2)
            nxt = i + 1

            @pl.when(nxt < cnt)
            def _():
                @pl.when(i >= 1)
                def _():
                    drain_out()
                issue(nxt, 1 - r)

            pltpu.make_async_copy(table_hbm.at[idx_v.at[r, 0]],
                                  buf_a.at[r], sem_a).wait()
            pltpu.make_async_copy(table_hbm.at[idx_v.at[r, 1]],
                                  buf_b.at[r], sem_b).wait()
            pltpu.make_async_copy(eproj_hbm.at[pl.ds(0, _CHUNK)],
                                  buf_e.at[r], sem_e).wait()

            def add_row(j, _):
                for kk in range(d // 16):
                    sl = pl.ds(kk * 16, 16)
                    buf_e[r, j, sl] = (buf_a[r, j, sl] + buf_b[r, j, sl]
                                       + buf_e[r, j, sl])
                return 0

            lax.fori_loop(0, _CHUNK, add_row, 0)
            pltpu.async_copy(buf_e.at[r],
                             out_hbm.at[pl.ds(cidf(i) * _CHUNK, _CHUNK)],
                             sem_o)
            return 0

        lax.fori_loop(0, cnt, body, 0)

        @pl.when(cnt >= 2)
        def _():
            drain_out()
        drain_out()

    return k(table, eproj, gidx)


def _sc_scatter(ne_list, cidx, n, nchunk_s):
    d = ne_list[0].shape[1]
    nslice = len(ne_list)
    info = plsc.get_sparse_core_info()
    nc, ns = info.num_cores, info.num_subcores
    per_core = nchunk_s // nc
    extra = nchunk_s - nc * per_core
    full, rem = per_core // ns, per_core % ns
    rows_per_sub = -(-n // (8 * ns)) * 8
    n_pad = rows_per_sub * ns
    last_rows = n - rows_per_sub * (ns - 1)
    assert last_rows > 0 and last_rows % 8 == 0
    mesh = plsc.VectorSubcoreMesh(core_axis_name="c", subcore_axis_name="s")

    @functools.partial(
        pl.kernel,
        out_type=jax.ShapeDtypeStruct((nc, n, d), jnp.float32),
        mesh=mesh,
        scratch_types=[
            pltpu.VMEM((1, _CHUNK), jnp.int32),
            pltpu.VMEM((_CHUNK, d), jnp.float32),
            pltpu.VMEM((8, d), jnp.float32),
            pltpu.VMEM_SHARED((n_pad, d), jnp.float32),
        ],
    )
    def k(*refs):
        ne_hbms = refs[:nslice]
        cidx_hbm, out_hbm, idx_v, ebuf, zbuf, shared = refs[nslice:]
        c = lax.axis_index("c")
        s = lax.axis_index("s")

        def zero_row(j, _):
            for kk in range(d // 16):
                zbuf[j, pl.ds(kk * 16, 16)] = jnp.zeros((16,), jnp.float32)
            return 0

        lax.fori_loop(0, 8, zero_row, 0)

        def zero_slice(i, _):
            pltpu.sync_copy(zbuf,
                            shared.at[pl.ds(s * rows_per_sub + i * 8, 8)])
            return 0

        lax.fori_loop(0, rows_per_sub // 8, zero_slice, 0)
        plsc.subcore_barrier()

        for sl, ne_hbm in enumerate(ne_hbms):
            chunk0 = sl * nchunk_s

            def do_chunk(cid, ne_hbm=ne_hbm, chunk0=chunk0):
                pltpu.sync_copy(cidx_hbm.at[chunk0 + cid], idx_v)
                pltpu.sync_copy(ne_hbm.at[pl.ds(cid * _CHUNK, _CHUNK)], ebuf)
                pltpu.sync_copy(ebuf, shared.at[idx_v.at[0]], add=True)

            def body(i, _, do_chunk=do_chunk):
                do_chunk(c * per_core + i * ns + s)
                return 0

            lax.fori_loop(0, full, body, 0)
            if rem:
                @pl.when(s < rem)
                def _(do_chunk=do_chunk):
                    do_chunk(c * per_core + full * ns + s)
            if extra:
                @pl.when((c == 0) & (s < extra))
                def _(do_chunk=do_chunk):
                    do_chunk(nc * per_core + s)
        plsc.subcore_barrier()

        @pl.when(s < ns - 1)
        def _():
            pltpu.sync_copy(
                shared.at[pl.ds(s * rows_per_sub, rows_per_sub)],
                out_hbm.at[c, pl.ds(s * rows_per_sub, rows_per_sub)])

        @pl.when(s == ns - 1)
        def _():
            pltpu.sync_copy(
                shared.at[pl.ds((ns - 1) * rows_per_sub, last_rows)],
                out_hbm.at[c, pl.ds((ns - 1) * rows_per_sub, last_rows)])

    return k(*ne_list, cidx)


def kernel(node_feat, edge_index, edge_feat, W_e1, b_e1, W_e2, b_e2,
           ln_e_g, ln_e_b, W_n1, b_n1, W_n2, b_n2, ln_n_g, ln_n_b):
    n, d = node_feat.shape
    e = edge_feat.shape[0]
    nchunk = e // _CHUNK
    e_s = e // _NSLICE
    nchunk_s = nchunk // _NSLICE
    block_e = 1000
    nblk_s = e_s // block_e
    row = edge_index[0]
    col = edge_index[1]

    gidx = jnp.stack([row.reshape(nchunk, _CHUNK),
                      col.reshape(nchunk, _CHUNK) + n], axis=1)
    cidx = col.reshape(nchunk, 1, _CHUNK)

    b_e1r = b_e1.reshape(1, d)
    b_e2r = b_e2.reshape(1, d)
    g_er = ln_e_g.reshape(1, d)
    be_er = ln_e_b.reshape(1, d)

    table = _proj_nodes(node_feat, W_e1[:2 * d])
    out_edge = jnp.zeros((e, d), jnp.float32)
    ne_list = []
    for i in range(_NSLICE):
        eproj_i = _proj_edges(edge_feat, W_e1[2 * d:], b_e1r,
                              block_e=block_e, blk0=i * nblk_s, nblk=nblk_s)
        hpre_i = _sc_gather(table, eproj_i, gidx,
                            chunk0=i * nchunk_s, nchunk=nchunk_s)
        ne_i, out_edge = _edge_mlp2(out_edge, hpre_i, edge_feat, W_e2,
                                    b_e2r, g_er, be_er, block_e=block_e,
                                    blk0=i * nblk_s, nblk=nblk_s)
        ne_list.append(ne_i)
    part = _sc_scatter(ne_list, cidx, n, nchunk_s=nchunk_s)

    new_node = _node_mlp(node_feat, [part], W_n1[:d], W_n1[d:],
                         b_n1.reshape(1, d), W_n2, b_n2.reshape(1, d),
                         ln_n_g.reshape(1, d), ln_n_b.reshape(1, d),
                         block_n=1000)
    return (new_node, out_edge)


# speedup vs baseline: 1.0398x; 1.0398x over previous
import functools

import jax
import jax.numpy as jnp
from jax import lax
from jax.experimental import pallas as pl
from jax.experimental.pallas import tpu as pltpu
from jax.experimental.pallas import tpu_sc as plsc

_CHUNK = 128
_NSLICE = 4


def _proj_nodes_body(nf_ref, w_ref, out_ref):
    out_ref[...] = jnp.dot(nf_ref[...], w_ref[...],
                           preferred_element_type=jnp.float32)


def _proj_nodes(nf, w12):
    n, d = nf.shape
    return pl.pallas_call(
        _proj_nodes_body,
        grid=(2,),
        in_specs=[
            pl.BlockSpec((n, d), lambda i: (0, 0)),
            pl.BlockSpec((d, d), lambda i: (i, 0)),
        ],
        out_specs=pl.BlockSpec((n, d), lambda i: (i, 0)),
        out_shape=jax.ShapeDtypeStruct((2 * n, d), jnp.float32),
    )(nf, w12)


def _proj_edges_body(ef_ref, w_ref, b_ref, out_ref):
    out_ref[...] = jnp.dot(ef_ref[...], w_ref[...],
                           preferred_element_type=jnp.float32) + b_ref[...]


def _proj_edges(ef, w, b, block_e, blk0, nblk):
    d = ef.shape[1]
    return pl.pallas_call(
        _proj_edges_body,
        grid=(nblk,),
        in_specs=[
            pl.BlockSpec((block_e, d), lambda i: (blk0 + i, 0)),
            pl.BlockSpec((d, d), lambda i: (0, 0)),
            pl.BlockSpec((1, d), lambda i: (0, 0)),
        ],
        out_specs=pl.BlockSpec((block_e, d), lambda i: (i, 0)),
        out_shape=jax.ShapeDtypeStruct((nblk * block_e, d), jnp.float32),
    )(ef, w, b)


def _edge_mlp2_body(of_ref, hpre_ref, ef_ref, w_ref, b_ref, g_ref, beta_ref,
                    ne_ref, out_ref):
    del of_ref
    h = jnp.maximum(hpre_ref[...], 0.0)
    y = jnp.dot(h, w_ref[...], preferred_element_type=jnp.float32) + b_ref[...]
    y = jnp.maximum(y, 0.0)
    mu = jnp.mean(y, axis=-1, keepdims=True)
    var = jnp.mean(jnp.square(y - mu), axis=-1, keepdims=True)
    ne = (y - mu) * jax.lax.rsqrt(var + 1e-5) * g_ref[...] + beta_ref[...]
    ne_ref[...] = ne
    out_ref[...] = ne + ef_ref[...]


def _edge_mlp2(of, hpre, ef, w, b, g, beta, block_e, blk0, nblk):
    e, d = ef.shape
    return pl.pallas_call(
        _edge_mlp2_body,
        grid=(nblk,),
        in_specs=[
            pl.BlockSpec(memory_space=pl.ANY),
            pl.BlockSpec((block_e, d), lambda i: (i, 0)),
            pl.BlockSpec((block_e, d), lambda i: (blk0 + i, 0)),
            pl.BlockSpec((d, d), lambda i: (0, 0)),
            pl.BlockSpec((1, d), lambda i: (0, 0)),
            pl.BlockSpec((1, d), lambda i: (0, 0)),
            pl.BlockSpec((1, d), lambda i: (0, 0)),
        ],
        out_specs=[
            pl.BlockSpec((block_e, d), lambda i: (i, 0)),
            pl.BlockSpec((block_e, d), lambda i: (blk0 + i, 0)),
        ],
        out_shape=[
            jax.ShapeDtypeStruct((nblk * block_e, d), jnp.float32),
            jax.ShapeDtypeStruct((e, d), jnp.float32),
        ],
        input_output_aliases={0: 1},
    )(of, hpre, ef, w, b, g, beta)


def _node_mlp_body(naps, nf_ref, *rest):
    ap_refs = rest[:naps]
    (w1a_ref, w1b_ref, b1_ref, w2_ref, b2_ref, g_ref, beta_ref,
     out_ref) = rest[naps:]
    nf = nf_ref[...]
    aggr = ap_refs[0][0] + ap_refs[0][1]
    for ap in ap_refs[1:]:
        aggr = aggr + ap[0] + ap[1]
    y = (jnp.dot(nf, w1a_ref[...], preferred_element_type=jnp.float32)
         + jnp.dot(aggr, w1b_ref[...], preferred_element_type=jnp.float32)
         + b1_ref[...])
    y = jnp.maximum(y, 0.0)
    y = jnp.dot(y, w2_ref[...], preferred_element_type=jnp.float32) + b2_ref[...]
    y = jnp.maximum(y, 0.0)
    mu = jnp.mean(y, axis=-1, keepdims=True)
    var = jnp.mean(jnp.square(y - mu), axis=-1, keepdims=True)
    out_ref[...] = ((y - mu) * jax.lax.rsqrt(var + 1e-5) * g_ref[...]
                    + beta_ref[...] + nf)


def _node_mlp(nf, aps, w1a, w1b, b1, w2, b2, g, beta, block_n):
    n, d = nf.shape
    nc = aps[0].shape[0]
    return pl.pallas_call(
        functools.partial(_node_mlp_body, len(aps)),
        grid=(n // block_n,),
        in_specs=[pl.BlockSpec((block_n, d), lambda i: (i, 0))]
        + [pl.BlockSpec((nc, block_n, d), lambda i: (0, i, 0))
           for _ in aps]
        + [
            pl.BlockSpec((d, d), lambda i: (0, 0)),
            pl.BlockSpec((d, d), lambda i: (0, 0)),
            pl.BlockSpec((1, d), lambda i: (0, 0)),
            pl.BlockSpec((d, d), lambda i: (0, 0)),
            pl.BlockSpec((1, d), lambda i: (0, 0)),
            pl.BlockSpec((1, d), lambda i: (0, 0)),
            pl.BlockSpec((1, d), lambda i: (0, 0)),
        ],
        out_specs=pl.BlockSpec((block_n, d), lambda i: (i, 0)),
        out_shape=jax.ShapeDtypeStruct((n, d), jnp.float32),
    )(nf, *aps, w1a, w1b, b1, w2, b2, g, beta)


def _sc_gather(table, eproj, gidx, chunk0, nchunk):
    es, d = eproj.shape
    info = plsc.get_sparse_core_info()
    nc, ns = info.num_cores, info.num_subcores
    nw = nc * ns
    full, rem = nchunk // nw, nchunk % nw
    mesh = plsc.VectorSubcoreMesh(core_axis_name="c", subcore_axis_name="s")

    assert full >= 2

    @functools.partial(
        pl.kernel,
        out_type=jax.ShapeDtypeStruct((es, d), jnp.float32),
        mesh=mesh,
        scratch_types=[
            pltpu.VMEM((2, 2, _CHUNK), jnp.int32),
            pltpu.VMEM((2, _CHUNK, d), jnp.float32),
            pltpu.VMEM((2, _CHUNK, d), jnp.float32),
            pltpu.VMEM((2, _CHUNK, d), jnp.float32),
            pltpu.SemaphoreType.DMA,
            pltpu.SemaphoreType.DMA,
            pltpu.SemaphoreType.DMA,
            pltpu.SemaphoreType.DMA,
        ],
    )
    def k(table_hbm, eproj_hbm, gidx_hbm, out_hbm,
          idx_v, buf_a, buf_bo, buf_e, sem_a, sem_b, sem_e, sem_o):
        cid0 = lax.axis_index("s") * nc + lax.axis_index("c")
        cnt = full + jnp.where(cid0 < rem, 1, 0) if rem else full

        def cidf(i):
            return i * nw + cid0

        def drain_out():
            pltpu.make_async_copy(
                buf_bo.at[0], out_hbm.at[pl.ds(0, _CHUNK)], sem_o).wait()

        def issue(i, r, drain):
            pltpu.sync_copy(gidx_hbm.at[chunk0 + cidf(i)], idx_v.at[r])
            pltpu.async_copy(table_hbm.at[idx_v.at[r, 0]], buf_a.at[r], sem_a)
            if drain:
                drain_out()
            pltpu.async_copy(table_hbm.at[idx_v.at[r, 1]], buf_bo.at[r], sem_b)
            pltpu.async_copy(
                eproj_hbm.at[pl.ds(cidf(i) * _CHUNK, _CHUNK)],
                buf_e.at[r], sem_e)

        def process(i, r):
            pltpu.make_async_copy(table_hbm.at[idx_v.at[r, 0]],
                                  buf_a.at[r], sem_a).wait()
            pltpu.make_async_copy(table_hbm.at[idx_v.at[r, 1]],
                                  buf_bo.at[r], sem_b).wait()
            pltpu.make_async_copy(eproj_hbm.at[pl.ds(0, _CHUNK)],
                                  buf_e.at[r], sem_e).wait()

            def add_row(j, _):
                for kk in range(d // 16):
                    sl = pl.ds(kk * 16, 16)
                    buf_bo[r, j, sl] = (buf_a[r, j, sl] + buf_bo[r, j, sl]
                                        + buf_e[r, j, sl])
                return 0

            lax.fori_loop(0, _CHUNK, add_row, 0)
            pltpu.async_copy(buf_bo.at[r],
                             out_hbm.at[pl.ds(cidf(i) * _CHUNK, _CHUNK)],
                             sem_o)

        issue(0, 0, drain=False)
        issue(1, 1, drain=False)

        def pair_body(p, _):
            base = 2 * p
            process(base, 0)

            @pl.when(base + 2 < cnt)
            def _():
                issue(base + 2, 0, drain=True)

            process(base + 1, 1)

            @pl.when(base + 3 < cnt)
            def _():
                issue(base + 3, 1, drain=True)
            return 0

        lax.fori_loop(0, cnt // 2, pair_body, 0)
        if rem or full % 2:
            @pl.when(lax.rem(cnt, 2) == 1)
            def _():
                process(cnt - 1, 0)
        drain_out()
        drain_out()

    return k(table, eproj, gidx)


def _sc_scatter(ne_list, cidx, n, nchunk_s):
    d = ne_list[0].shape[1]
    nslice = len(ne_list)
    info = plsc.get_sparse_core_info()
    nc, ns = info.num_cores, info.num_subcores
    per_core = nchunk_s // nc
    extra = nchunk_s - nc * per_core
    full, rem = per_core // ns, per_core % ns
    rows_per_sub = -(-n // (8 * ns)) * 8
    n_pad = rows_per_sub * ns
    last_rows = n - rows_per_sub * (ns - 1)
    assert last_rows > 0 and last_rows % 8 == 0
    mesh = plsc.VectorSubcoreMesh(core_axis_name="c", subcore_axis_name="s")

    @functools.partial(
        pl.kernel,
        out_type=jax.ShapeDtypeStruct((nc, n, d), jnp.float32),
        mesh=mesh,
        scratch_types=[
            pltpu.VMEM((1, _CHUNK), jnp.int32),
            pltpu.VMEM((_CHUNK, d), jnp.float32),
            pltpu.VMEM((8, d), jnp.float32),
            pltpu.VMEM_SHARED((n_pad, d), jnp.float32),
        ],
    )
    def k(*refs):
        ne_hbms = refs[:nslice]
        cidx_hbm, out_hbm, idx_v, ebuf, zbuf, shared = refs[nslice:]
        c = lax.axis_index("c")
        s = lax.axis_index("s")

        def zero_row(j, _):
            for kk in range(d // 16):
                zbuf[j, pl.ds(kk * 16, 16)] = jnp.zeros((16,), jnp.float32)
            return 0

        lax.fori_loop(0, 8, zero_row, 0)

        def zero_slice(i, _):
            pltpu.sync_copy(zbuf,
                            shared.at[pl.ds(s * rows_per_sub + i * 8, 8)])
            return 0

        lax.fori_loop(0, rows_per_sub // 8, zero_slice, 0)
        plsc.subcore_barrier()

        for sl, ne_hbm in enumerate(ne_hbms):
            chunk0 = sl * nchunk_s

            def do_chunk(cid, ne_hbm=ne_hbm, chunk0=chunk0):
                pltpu.sync_copy(cidx_hbm.at[chunk0 + cid], idx_v)
                pltpu.sync_copy(ne_hbm.at[pl.ds(cid * _CHUNK, _CHUNK)], ebuf)
                pltpu.sync_copy(ebuf, shared.at[idx_v.at[0]], add=True)

            def body(i, _, do_chunk=do_chunk):
                do_chunk(c * per_core + i * ns + s)
                return 0

            lax.fori_loop(0, full, body, 0)
            if rem:
                @pl.when(s < rem)
                def _(do_chunk=do_chunk):
                    do_chunk(c * per_core + full * ns + s)
            if extra:
                @pl.when((c == 0) & (s < extra))
                def _(do_chunk=do_chunk):
                    do_chunk(nc * per_core + s)
        plsc.subcore_barrier()

        @pl.when(s < ns - 1)
        def _():
            pltpu.sync_copy(
                shared.at[pl.ds(s * rows_per_sub, rows_per_sub)],
                out_hbm.at[c, pl.ds(s * rows_per_sub, rows_per_sub)])

        @pl.when(s == ns - 1)
        def _():
            pltpu.sync_copy(
                shared.at[pl.ds((ns - 1) * rows_per_sub, last_rows)],
                out_hbm.at[c, pl.ds((ns - 1) * rows_per_sub, last_rows)])

    return k(*ne_list, cidx)


def kernel(node_feat, edge_index, edge_feat, W_e1, b_e1, W_e2, b_e2,
           ln_e_g, ln_e_b, W_n1, b_n1, W_n2, b_n2, ln_n_g, ln_n_b):
    n, d = node_feat.shape
    e = edge_feat.shape[0]
    nchunk = e // _CHUNK
    e_s = e // _NSLICE
    nchunk_s = nchunk // _NSLICE
    block_e = 1000
    nblk_s = e_s // block_e
    row = edge_index[0]
    col = edge_index[1]

    gidx = jnp.stack([row.reshape(nchunk, _CHUNK),
                      col.reshape(nchunk, _CHUNK) + n], axis=1)
    cidx = col.reshape(nchunk, 1, _CHUNK)

    b_e1r = b_e1.reshape(1, d)
    b_e2r = b_e2.reshape(1, d)
    g_er = ln_e_g.reshape(1, d)
    be_er = ln_e_b.reshape(1, d)

    table = _proj_nodes(node_feat, W_e1[:2 * d])
    out_edge = jnp.zeros((e, d), jnp.float32)
    ne_list = []
    for i in range(_NSLICE):
        eproj_i = _proj_edges(edge_feat, W_e1[2 * d:], b_e1r,
                              block_e=block_e, blk0=i * nblk_s, nblk=nblk_s)
        hpre_i = _sc_gather(table, eproj_i, gidx,
                            chunk0=i * nchunk_s, nchunk=nchunk_s)
        ne_i, out_edge = _edge_mlp2(out_edge, hpre_i, edge_feat, W_e2,
                                    b_e2r, g_er, be_er, block_e=block_e,
                                    blk0=i * nblk_s, nblk=nblk_s)
        ne_list.append(ne_i)
    part = _sc_scatter(ne_list, cidx, n, nchunk_s=nchunk_s)

    new_node = _node_mlp(node_feat, [part], W_n1[:d], W_n1[d:],
                         b_n1.reshape(1, d), W_n2, b_n2.reshape(1, d),
                         ln_n_g.reshape(1, d), ln_n_b.reshape(1, d),
                         block_n=1000)
    return (new_node, out_edge)


# speedup vs baseline: 1.2151x; 1.1687x over previous
import functools

import jax
import jax.numpy as jnp
from jax import lax
from jax.experimental import pallas as pl
from jax.experimental.pallas import tpu as pltpu
from jax.experimental.pallas import tpu_sc as plsc

_CHUNK = 128
_NSLICE = 4


def _proj_nodes_body(nf_ref, w_ref, out_ref):
    out_ref[...] = jnp.dot(nf_ref[...], w_ref[...],
                           preferred_element_type=jnp.float32)


def _proj_nodes(nf, w12):
    n, d = nf.shape
    return pl.pallas_call(
        _proj_nodes_body,
        grid=(2,),
        in_specs=[
            pl.BlockSpec((n, d), lambda i: (0, 0)),
            pl.BlockSpec((d, d), lambda i: (i, 0)),
        ],
        out_specs=pl.BlockSpec((n, d), lambda i: (i, 0)),
        out_shape=jax.ShapeDtypeStruct((2 * n, d), jnp.float32),
    )(nf, w12)


def _proj_edges_body(ef_ref, w_ref, b_ref, out_ref):
    out_ref[...] = jnp.dot(ef_ref[...], w_ref[...],
                           preferred_element_type=jnp.float32) + b_ref[...]


def _proj_edges(ef, w, b, block_e, blk0, nblk):
    d = ef.shape[1]
    return pl.pallas_call(
        _proj_edges_body,
        grid=(nblk,),
        in_specs=[
            pl.BlockSpec((block_e, d), lambda i: (blk0 + i, 0)),
            pl.BlockSpec((d, d), lambda i: (0, 0)),
            pl.BlockSpec((1, d), lambda i: (0, 0)),
        ],
        out_specs=pl.BlockSpec((block_e, d), lambda i: (i, 0)),
        out_shape=jax.ShapeDtypeStruct((nblk * block_e, d), jnp.float32),
    )(ef, w, b)


def _edge_mlp2_body(of_ref, hpre_ref, ef_ref, w_ref, b_ref, g_ref, beta_ref,
                    ne_ref, out_ref):
    del of_ref
    h = jnp.maximum(hpre_ref[...], 0.0)
    y = jnp.dot(h, w_ref[...], preferred_element_type=jnp.float32) + b_ref[...]
    y = jnp.maximum(y, 0.0)
    mu = jnp.mean(y, axis=-1, keepdims=True)
    var = jnp.mean(jnp.square(y - mu), axis=-1, keepdims=True)
    ne = (y - mu) * jax.lax.rsqrt(var + 1e-5) * g_ref[...] + beta_ref[...]
    ne_ref[...] = ne
    out_ref[...] = ne + ef_ref[...]


def _edge_mlp2(of, hpre, ef, w, b, g, beta, block_e, blk0, nblk):
    e, d = ef.shape
    return pl.pallas_call(
        _edge_mlp2_body,
        grid=(nblk,),
        in_specs=[
            pl.BlockSpec(memory_space=pl.ANY),
            pl.BlockSpec((block_e, d), lambda i: (i, 0)),
            pl.BlockSpec((block_e, d), lambda i: (blk0 + i, 0)),
            pl.BlockSpec((d, d), lambda i: (0, 0)),
            pl.BlockSpec((1, d), lambda i: (0, 0)),
            pl.BlockSpec((1, d), lambda i: (0, 0)),
            pl.BlockSpec((1, d), lambda i: (0, 0)),
        ],
        out_specs=[
            pl.BlockSpec((block_e, d), lambda i: (i, 0)),
            pl.BlockSpec((block_e, d), lambda i: (blk0 + i, 0)),
        ],
        out_shape=[
            jax.ShapeDtypeStruct((nblk * block_e, d), jnp.float32),
            jax.ShapeDtypeStruct((e, d), jnp.float32),
        ],
        input_output_aliases={0: 1},
    )(of, hpre, ef, w, b, g, beta)


def _node_mlp_body(naps, nf_ref, *rest):
    ap_refs = rest[:naps]
    (w1a_ref, w1b_ref, b1_ref, w2_ref, b2_ref, g_ref, beta_ref,
     out_ref) = rest[naps:]
    nf = nf_ref[...]
    aggr = ap_refs[0][0] + ap_refs[0][1]
    for ap in ap_refs[1:]:
        aggr = aggr + ap[0] + ap[1]
    y = (jnp.dot(nf, w1a_ref[...], preferred_element_type=jnp.float32)
         + jnp.dot(aggr, w1b_ref[...], preferred_element_type=jnp.float32)
         + b1_ref[...])
    y = jnp.maximum(y, 0.0)
    y = jnp.dot(y, w2_ref[...], preferred_element_type=jnp.float32) + b2_ref[...]
    y = jnp.maximum(y, 0.0)
    mu = jnp.mean(y, axis=-1, keepdims=True)
    var = jnp.mean(jnp.square(y - mu), axis=-1, keepdims=True)
    out_ref[...] = ((y - mu) * jax.lax.rsqrt(var + 1e-5) * g_ref[...]
                    + beta_ref[...] + nf)


def _node_mlp(nf, aps, w1a, w1b, b1, w2, b2, g, beta, block_n):
    n, d = nf.shape
    nc = aps[0].shape[0]
    return pl.pallas_call(
        functools.partial(_node_mlp_body, len(aps)),
        grid=(n // block_n,),
        in_specs=[pl.BlockSpec((block_n, d), lambda i: (i, 0))]
        + [pl.BlockSpec((nc, block_n, d), lambda i: (0, i, 0))
           for _ in aps]
        + [
            pl.BlockSpec((d, d), lambda i: (0, 0)),
            pl.BlockSpec((d, d), lambda i: (0, 0)),
            pl.BlockSpec((1, d), lambda i: (0, 0)),
            pl.BlockSpec((d, d), lambda i: (0, 0)),
            pl.BlockSpec((1, d), lambda i: (0, 0)),
            pl.BlockSpec((1, d), lambda i: (0, 0)),
            pl.BlockSpec((1, d), lambda i: (0, 0)),
        ],
        out_specs=pl.BlockSpec((block_n, d), lambda i: (i, 0)),
        out_shape=jax.ShapeDtypeStruct((n, d), jnp.float32),
    )(nf, *aps, w1a, w1b, b1, w2, b2, g, beta)


def _sc_gather(table, eproj, gidx, chunk0, nchunk):
    es, d = eproj.shape
    info = plsc.get_sparse_core_info()
    nc, ns = info.num_cores, info.num_subcores
    nw = nc * ns
    full, rem = nchunk // nw, nchunk % nw
    mesh = plsc.VectorSubcoreMesh(core_axis_name="c", subcore_axis_name="s")

    assert full >= 2

    @functools.partial(
        pl.kernel,
        out_type=jax.ShapeDtypeStruct((es, d), jnp.float32),
        mesh=mesh,
        scratch_types=[
            pltpu.VMEM((2, 2, _CHUNK), jnp.int32),
            pltpu.VMEM((2, _CHUNK, d), jnp.float32),
            pltpu.VMEM((2, _CHUNK, d), jnp.float32),
            pltpu.VMEM((2, _CHUNK, d), jnp.float32),
            pltpu.SemaphoreType.DMA,
            pltpu.SemaphoreType.DMA,
            pltpu.SemaphoreType.DMA,
            pltpu.SemaphoreType.DMA,
        ],
    )
    def k(table_hbm, eproj_hbm, gidx_hbm, out_hbm,
          idx_v, buf_a, buf_bo, buf_e, sem_a, sem_b, sem_e, sem_o):
        cid0 = lax.axis_index("s") * nc + lax.axis_index("c")
        cnt = full + jnp.where(cid0 < rem, 1, 0) if rem else full

        def cidf(i):
            return i * nw + cid0

        def drain_out():
            pltpu.make_async_copy(
                buf_bo.at[0], out_hbm.at[pl.ds(0, _CHUNK)], sem_o).wait()

        def issue(i, r, drain):
            pltpu.sync_copy(gidx_hbm.at[chunk0 + cidf(i)], idx_v.at[r])
            pltpu.async_copy(table_hbm.at[idx_v.at[r, 0]], buf_a.at[r], sem_a)
            if drain:
                drain_out()
            pltpu.async_copy(table_hbm.at[idx_v.at[r, 1]], buf_bo.at[r], sem_b)
            pltpu.async_copy(
                eproj_hbm.at[pl.ds(cidf(i) * _CHUNK, _CHUNK)],
                buf_e.at[r], sem_e)

        def process(i, r):
            pltpu.make_async_copy(table_hbm.at[idx_v.at[r, 0]],
                                  buf_a.at[r], sem_a).wait()
            pltpu.make_async_copy(table_hbm.at[idx_v.at[r, 1]],
                                  buf_bo.at[r], sem_b).wait()
            pltpu.make_async_copy(eproj_hbm.at[pl.ds(0, _CHUNK)],
                                  buf_e.at[r], sem_e).wait()

            def add_row(j, _):
                for kk in range(d // 16):
                    sl = pl.ds(kk * 16, 16)
                    buf_bo[r, j, sl] = (buf_a[r, j, sl] + buf_bo[r, j, sl]
                                        + buf_e[r, j, sl])
                return 0

            lax.fori_loop(0, _CHUNK, add_row, 0)
            pltpu.async_copy(buf_bo.at[r],
                             out_hbm.at[pl.ds(cidf(i) * _CHUNK, _CHUNK)],
                             sem_o)

        issue(0, 0, drain=False)
        issue(1, 1, drain=False)

        def pair_body(p, _):
            base = 2 * p
            process(base, 0)

            @pl.when(base + 2 < cnt)
            def _():
                issue(base + 2, 0, drain=True)

            process(base + 1, 1)

            @pl.when(base + 3 < cnt)
            def _():
                issue(base + 3, 1, drain=True)
            return 0

        lax.fori_loop(0, cnt // 2, pair_body, 0)
        if rem or full % 2:
            @pl.when(lax.rem(cnt, 2) == 1)
            def _():
                process(cnt - 1, 0)
        drain_out()
        drain_out()

    return k(table, eproj, gidx)


def _sc_scatter(ne, cidx, n, chunk0, nchunk):
    d = ne.shape[1]
    info = plsc.get_sparse_core_info()
    nc, ns = info.num_cores, info.num_subcores
    nw = nc * ns
    full, rem = nchunk // nw, nchunk % nw
    assert full >= 2
    rows_per_sub = -(-n // (8 * ns)) * 8
    n_pad = rows_per_sub * ns
    last_rows = n - rows_per_sub * (ns - 1)
    assert last_rows > 0 and last_rows % 8 == 0
    mesh = plsc.VectorSubcoreMesh(core_axis_name="c", subcore_axis_name="s")

    @functools.partial(
        pl.kernel,
        out_type=jax.ShapeDtypeStruct((nc, n, d), jnp.float32),
        mesh=mesh,
        scratch_types=[
            pltpu.VMEM((2, 1, _CHUNK), jnp.int32),
            pltpu.VMEM((2, _CHUNK, d), jnp.float32),
            pltpu.VMEM((8, d), jnp.float32),
            pltpu.VMEM_SHARED((n_pad, d), jnp.float32),
            pltpu.SemaphoreType.DMA,
        ],
    )
    def k(ne_hbm, cidx_hbm, out_hbm, idx_v, ebuf, zbuf, shared, sem_r):
        c = lax.axis_index("c")
        s = lax.axis_index("s")
        cid0 = s * nc + c
        cnt = full + jnp.where(cid0 < rem, 1, 0) if rem else full

        def cidf(i):
            return i * nw + cid0

        def zero_row(j, _):
            for kk in range(d // 16):
                zbuf[j, pl.ds(kk * 16, 16)] = jnp.zeros((16,), jnp.float32)
            return 0

        lax.fori_loop(0, 8, zero_row, 0)

        def zero_slice(i, _):
            pltpu.sync_copy(zbuf,
                            shared.at[pl.ds(s * rows_per_sub + i * 8, 8)])
            return 0

        lax.fori_loop(0, rows_per_sub // 8, zero_slice, 0)
        plsc.subcore_barrier()

        def issue(i, r):
            pltpu.async_copy(cidx_hbm.at[chunk0 + cidf(i)], idx_v.at[r],
                             sem_r)
            pltpu.async_copy(ne_hbm.at[pl.ds(cidf(i) * _CHUNK, _CHUNK)],
                             ebuf.at[r], sem_r)

        def process(i, r):
            pltpu.make_async_copy(cidx_hbm.at[chunk0], idx_v.at[r],
                                  sem_r).wait()
            pltpu.make_async_copy(ne_hbm.at[pl.ds(0, _CHUNK)], ebuf.at[r],
                                  sem_r).wait()
            pltpu.sync_copy(ebuf.at[r], shared.at[idx_v.at[r, 0]], add=True)

        issue(0, 0)
        issue(1, 1)

        def pair_body(p, _):
            base = 2 * p
            process(base, 0)

            @pl.when(base + 2 < cnt)
            def _():
                issue(base + 2, 0)

            process(base + 1, 1)

            @pl.when(base + 3 < cnt)
            def _():
                issue(base + 3, 1)
            return 0

        lax.fori_loop(0, cnt // 2, pair_body, 0)
        if rem or full % 2:
            @pl.when(lax.rem(cnt, 2) == 1)
            def _():
                process(cnt - 1, 0)
        plsc.subcore_barrier()

        @pl.when(s < ns - 1)
        def _():
            pltpu.sync_copy(
                shared.at[pl.ds(s * rows_per_sub, rows_per_sub)],
                out_hbm.at[c, pl.ds(s * rows_per_sub, rows_per_sub)])

        @pl.when(s == ns - 1)
        def _():
            pltpu.sync_copy(
                shared.at[pl.ds((ns - 1) * rows_per_sub, last_rows)],
                out_hbm.at[c, pl.ds((ns - 1) * rows_per_sub, last_rows)])

    return k(ne, cidx)


def kernel(node_feat, edge_index, edge_feat, W_e1, b_e1, W_e2, b_e2,
           ln_e_g, ln_e_b, W_n1, b_n1, W_n2, b_n2, ln_n_g, ln_n_b):
    n, d = node_feat.shape
    e = edge_feat.shape[0]
    nchunk = e // _CHUNK
    e_s = e // _NSLICE
    nchunk_s = nchunk // _NSLICE
    block_e = 1000
    nblk_s = e_s // block_e
    row = edge_index[0]
    col = edge_index[1]

    gidx = jnp.stack([row.reshape(nchunk, _CHUNK),
                      col.reshape(nchunk, _CHUNK) + n], axis=1)
    cidx = col.reshape(nchunk, 1, _CHUNK)

    b_e1r = b_e1.reshape(1, d)
    b_e2r = b_e2.reshape(1, d)
    g_er = ln_e_g.reshape(1, d)
    be_er = ln_e_b.reshape(1, d)

    table = _proj_nodes(node_feat, W_e1[:2 * d])
    out_edge = jnp.zeros((e, d), jnp.float32)
    ne_list = []
    for i in range(_NSLICE):
        eproj_i = _proj_edges(edge_feat, W_e1[2 * d:], b_e1r,
                              block_e=block_e, blk0=i * nblk_s, nblk=nblk_s)
        hpre_i = _sc_gather(table, eproj_i, gidx,
                            chunk0=i * nchunk_s, nchunk=nchunk_s)
        ne_i, out_edge = _edge_mlp2(out_edge, hpre_i, edge_feat, W_e2,
                                    b_e2r, g_er, be_er, block_e=block_e,
                                    blk0=i * nblk_s, nblk=nblk_s)
        ne_list.append(ne_i)
    parts = [_sc_scatter(ne_i, cidx, n, chunk0=i * nchunk_s,
                         nchunk=nchunk_s)
             for i, ne_i in enumerate(ne_list)]

    new_node = _node_mlp(node_feat, parts, W_n1[:d], W_n1[d:],
                         b_n1.reshape(1, d), W_n2, b_n2.reshape(1, d),
                         ln_n_g.reshape(1, d), ln_n_b.reshape(1, d),
                         block_n=1000)
    return (new_node, out_edge)


# speedup vs baseline: 1.5104x; 1.2429x over previous
import functools

import jax
import jax.numpy as jnp
from jax import lax
from jax.experimental import pallas as pl
from jax.experimental.pallas import tpu as pltpu
from jax.experimental.pallas import tpu_sc as plsc

_CHUNK = 128
_NSLICE = 4


def _proj_nodes_body(nf_ref, w_ref, out_ref):
    out_ref[...] = jnp.dot(nf_ref[...], w_ref[...],
                           preferred_element_type=jnp.float32)


def _proj_nodes(nf, w12):
    n, d = nf.shape
    return pl.pallas_call(
        _proj_nodes_body,
        grid=(2,),
        in_specs=[
            pl.BlockSpec((n, d), lambda i: (0, 0)),
            pl.BlockSpec((d, d), lambda i: (i, 0)),
        ],
        out_specs=pl.BlockSpec((n, d), lambda i: (i, 0)),
        out_shape=jax.ShapeDtypeStruct((2 * n, d), jnp.float32),
    )(nf, w12)


def _proj_edges_body(ef_ref, w_ref, b_ref, out_ref):
    out_ref[...] = jnp.dot(ef_ref[...], w_ref[...],
                           preferred_element_type=jnp.float32) + b_ref[...]


def _proj_edges(ef, w, b, block_e, blk0, nblk):
    d = ef.shape[1]
    return pl.pallas_call(
        _proj_edges_body,
        grid=(nblk,),
        in_specs=[
            pl.BlockSpec((block_e, d), lambda i: (blk0 + i, 0)),
            pl.BlockSpec((d, d), lambda i: (0, 0)),
            pl.BlockSpec((1, d), lambda i: (0, 0)),
        ],
        out_specs=pl.BlockSpec((block_e, d), lambda i: (i, 0)),
        out_shape=jax.ShapeDtypeStruct((nblk * block_e, d), jnp.float32),
    )(ef, w, b)


def _edge_mlp2_body(of_ref, hpre_ref, ef_ref, w_ref, b_ref, g_ref, beta_ref,
                    ne_ref, out_ref):
    del of_ref
    h = jnp.maximum(hpre_ref[...], 0.0)
    y = jnp.dot(h, w_ref[...], preferred_element_type=jnp.float32) + b_ref[...]
    y = jnp.maximum(y, 0.0)
    mu = jnp.mean(y, axis=-1, keepdims=True)
    var = jnp.mean(jnp.square(y - mu), axis=-1, keepdims=True)
    ne = (y - mu) * jax.lax.rsqrt(var + 1e-5) * g_ref[...] + beta_ref[...]
    ne_ref[...] = ne
    out_ref[...] = ne + ef_ref[...]


def _edge_mlp2(of, hpre, ef, w, b, g, beta, block_e, blk0, nblk):
    e, d = ef.shape
    first = of is None
    args = (hpre, ef, w, b, g, beta) if first else (of, hpre, ef, w, b, g,
                                                   beta)
    body = (functools.partial(_edge_mlp2_body, None) if first
            else _edge_mlp2_body)
    return pl.pallas_call(
        body,
        grid=(nblk,),
        in_specs=([] if first else [pl.BlockSpec(memory_space=pl.ANY)]) + [
            pl.BlockSpec((block_e, d), lambda i: (i, 0)),
            pl.BlockSpec((block_e, d), lambda i: (blk0 + i, 0)),
            pl.BlockSpec((d, d), lambda i: (0, 0)),
            pl.BlockSpec((1, d), lambda i: (0, 0)),
            pl.BlockSpec((1, d), lambda i: (0, 0)),
            pl.BlockSpec((1, d), lambda i: (0, 0)),
        ],
        out_specs=[
            pl.BlockSpec((block_e, d), lambda i: (i, 0)),
            pl.BlockSpec((block_e, d), lambda i: (blk0 + i, 0)),
        ],
        out_shape=[
            jax.ShapeDtypeStruct((nblk * block_e, d), jnp.float32),
            jax.ShapeDtypeStruct((e, d), jnp.float32),
        ],
        **({} if first else {"input_output_aliases": {0: 1}}),
    )(*args)


def _node_mlp_body(naps, nf_ref, *rest):
    ap_refs = rest[:naps]
    (w1a_ref, w1b_ref, b1_ref, w2_ref, b2_ref, g_ref, beta_ref,
     out_ref) = rest[naps:]
    nf = nf_ref[...]
    aggr = ap_refs[0][0] + ap_refs[0][1]
    for ap in ap_refs[1:]:
        aggr = aggr + ap[0] + ap[1]
    y = (jnp.dot(nf, w1a_ref[...], preferred_element_type=jnp.float32)
         + jnp.dot(aggr, w1b_ref[...], preferred_element_type=jnp.float32)
         + b1_ref[...])
    y = jnp.maximum(y, 0.0)
    y = jnp.dot(y, w2_ref[...], preferred_element_type=jnp.float32) + b2_ref[...]
    y = jnp.maximum(y, 0.0)
    mu = jnp.mean(y, axis=-1, keepdims=True)
    var = jnp.mean(jnp.square(y - mu), axis=-1, keepdims=True)
    out_ref[...] = ((y - mu) * jax.lax.rsqrt(var + 1e-5) * g_ref[...]
                    + beta_ref[...] + nf)


def _node_mlp(nf, aps, w1a, w1b, b1, w2, b2, g, beta, block_n):
    n, d = nf.shape
    nc = aps[0].shape[0]
    return pl.pallas_call(
        functools.partial(_node_mlp_body, len(aps)),
        grid=(n // block_n,),
        in_specs=[pl.BlockSpec((block_n, d), lambda i: (i, 0))]
        + [pl.BlockSpec((nc, block_n, d), lambda i: (0, i, 0))
           for _ in aps]
        + [
            pl.BlockSpec((d, d), lambda i: (0, 0)),
            pl.BlockSpec((d, d), lambda i: (0, 0)),
            pl.BlockSpec((1, d), lambda i: (0, 0)),
            pl.BlockSpec((d, d), lambda i: (0, 0)),
            pl.BlockSpec((1, d), lambda i: (0, 0)),
            pl.BlockSpec((1, d), lambda i: (0, 0)),
            pl.BlockSpec((1, d), lambda i: (0, 0)),
        ],
        out_specs=pl.BlockSpec((block_n, d), lambda i: (i, 0)),
        out_shape=jax.ShapeDtypeStruct((n, d), jnp.float32),
    )(nf, *aps, w1a, w1b, b1, w2, b2, g, beta)


def _sc_gather(table, eproj, gidx, chunk0, nchunk):
    es, d = eproj.shape
    info = plsc.get_sparse_core_info()
    nc, ns = info.num_cores, info.num_subcores
    nw = nc * ns
    full, rem = nchunk // nw, nchunk % nw
    mesh = plsc.VectorSubcoreMesh(core_axis_name="c", subcore_axis_name="s")

    assert full >= 2

    @functools.partial(
        pl.kernel,
        out_type=jax.ShapeDtypeStruct((es, d), jnp.float32),
        mesh=mesh,
        scratch_types=[
            pltpu.VMEM((2, 2, _CHUNK), jnp.int32),
            pltpu.VMEM((2, _CHUNK, d), jnp.float32),
            pltpu.VMEM((2, _CHUNK, d), jnp.float32),
            pltpu.VMEM((2, _CHUNK, d), jnp.float32),
            pltpu.SemaphoreType.DMA,
            pltpu.SemaphoreType.DMA,
            pltpu.SemaphoreType.DMA,
            pltpu.SemaphoreType.DMA,
        ],
    )
    def k(table_hbm, eproj_hbm, gidx_hbm, out_hbm,
          idx_v, buf_a, buf_bo, buf_e, sem_a, sem_b, sem_e, sem_o):
        cid0 = lax.axis_index("s") * nc + lax.axis_index("c")
        cnt = full + jnp.where(cid0 < rem, 1, 0) if rem else full

        def cidf(i):
            return i * nw + cid0

        def drain_out():
            pltpu.make_async_copy(
                buf_bo.at[0], out_hbm.at[pl.ds(0, _CHUNK)], sem_o).wait()

        def issue(i, r, drain):
            pltpu.sync_copy(gidx_hbm.at[chunk0 + cidf(i)], idx_v.at[r])
            pltpu.async_copy(table_hbm.at[idx_v.at[r, 0]], buf_a.at[r], sem_a)
            if drain:
                drain_out()
            pltpu.async_copy(table_hbm.at[idx_v.at[r, 1]], buf_bo.at[r], sem_b)
            pltpu.async_copy(
                eproj_hbm.at[pl.ds(cidf(i) * _CHUNK, _CHUNK)],
                buf_e.at[r], sem_e)

        def process(i, r):
            pltpu.make_async_copy(table_hbm.at[idx_v.at[r, 0]],
                                  buf_a.at[r], sem_a).wait()
            pltpu.make_async_copy(table_hbm.at[idx_v.at[r, 1]],
                                  buf_bo.at[r], sem_b).wait()
            pltpu.make_async_copy(eproj_hbm.at[pl.ds(0, _CHUNK)],
                                  buf_e.at[r], sem_e).wait()

            def add_row(j, _):
                for kk in range(d // 16):
                    sl = pl.ds(kk * 16, 16)
                    buf_bo[r, j, sl] = (buf_a[r, j, sl] + buf_bo[r, j, sl]
                                        + buf_e[r, j, sl])
                return 0

            lax.fori_loop(0, _CHUNK, add_row, 0)
            pltpu.async_copy(buf_bo.at[r],
                             out_hbm.at[pl.ds(cidf(i) * _CHUNK, _CHUNK)],
                             sem_o)

        issue(0, 0, drain=False)
        issue(1, 1, drain=False)

        def pair_body(p, _):
            base = 2 * p
            process(base, 0)

            @pl.when(base + 2 < cnt)
            def _():
                issue(base + 2, 0, drain=True)

            process(base + 1, 1)

            @pl.when(base + 3 < cnt)
            def _():
                issue(base + 3, 1, drain=True)
            return 0

        lax.fori_loop(0, cnt // 2, pair_body, 0)
        if rem or full % 2:
            @pl.when(lax.rem(cnt, 2) == 1)
            def _():
                process(cnt - 1, 0)
        drain_out()
        drain_out()

    return k(table, eproj, gidx)


def _sc_scatter(ne, cidx, n, chunk0, nchunk):
    d = ne.shape[1]
    info = plsc.get_sparse_core_info()
    nc, ns = info.num_cores, info.num_subcores
    nw = nc * ns
    full, rem = nchunk // nw, nchunk % nw
    assert full >= 2
    rows_per_sub = -(-n // (8 * ns)) * 8
    n_pad = rows_per_sub * ns
    last_rows = n - rows_per_sub * (ns - 1)
    assert last_rows > 0 and last_rows % 8 == 0
    mesh = plsc.VectorSubcoreMesh(core_axis_name="c", subcore_axis_name="s")

    @functools.partial(
        pl.kernel,
        out_type=jax.ShapeDtypeStruct((nc, n, d), jnp.float32),
        mesh=mesh,
        scratch_types=[
            pltpu.VMEM((2, 1, _CHUNK), jnp.int32),
            pltpu.VMEM((2, _CHUNK, d), jnp.float32),
            pltpu.VMEM((8, d), jnp.float32),
            pltpu.VMEM_SHARED((n_pad, d), jnp.float32),
            pltpu.SemaphoreType.DMA,
        ],
    )
    def k(ne_hbm, cidx_hbm, out_hbm, idx_v, ebuf, zbuf, shared, sem_r):
        c = lax.axis_index("c")
        s = lax.axis_index("s")
        cid0 = s * nc + c
        cnt = full + jnp.where(cid0 < rem, 1, 0) if rem else full

        def cidf(i):
            return i * nw + cid0

        def zero_row(j, _):
            for kk in range(d // 16):
                zbuf[j, pl.ds(kk * 16, 16)] = jnp.zeros((16,), jnp.float32)
            return 0

        lax.fori_loop(0, 8, zero_row, 0)

        def zero_slice(i, _):
            pltpu.sync_copy(zbuf,
                            shared.at[pl.ds(s * rows_per_sub + i * 8, 8)])
            return 0

        lax.fori_loop(0, rows_per_sub // 8, zero_slice, 0)
        plsc.subcore_barrier()

        def issue(i, r):
            pltpu.async_copy(cidx_hbm.at[chunk0 + cidf(i)], idx_v.at[r],
                             sem_r)
            pltpu.async_copy(ne_hbm.at[pl.ds(cidf(i) * _CHUNK, _CHUNK)],
                             ebuf.at[r], sem_r)

        def process(i, r):
            pltpu.make_async_copy(cidx_hbm.at[chunk0], idx_v.at[r],
                                  sem_r).wait()
            pltpu.make_async_copy(ne_hbm.at[pl.ds(0, _CHUNK)], ebuf.at[r],
                                  sem_r).wait()
            pltpu.sync_copy(ebuf.at[r], shared.at[idx_v.at[r, 0]], add=True)

        issue(0, 0)
        issue(1, 1)

        def pair_body(p, _):
            base = 2 * p
            process(base, 0)

            @pl.when(base + 2 < cnt)
            def _():
                issue(base + 2, 0)

            process(base + 1, 1)

            @pl.when(base + 3 < cnt)
            def _():
                issue(base + 3, 1)
            return 0

        lax.fori_loop(0, cnt // 2, pair_body, 0)
        if rem or full % 2:
            @pl.when(lax.rem(cnt, 2) == 1)
            def _():
                process(cnt - 1, 0)
        plsc.subcore_barrier()

        @pl.when(s < ns - 1)
        def _():
            pltpu.sync_copy(
                shared.at[pl.ds(s * rows_per_sub, rows_per_sub)],
                out_hbm.at[c, pl.ds(s * rows_per_sub, rows_per_sub)])

        @pl.when(s == ns - 1)
        def _():
            pltpu.sync_copy(
                shared.at[pl.ds((ns - 1) * rows_per_sub, last_rows)],
                out_hbm.at[c, pl.ds((ns - 1) * rows_per_sub, last_rows)])

    return k(ne, cidx)


def kernel(node_feat, edge_index, edge_feat, W_e1, b_e1, W_e2, b_e2,
           ln_e_g, ln_e_b, W_n1, b_n1, W_n2, b_n2, ln_n_g, ln_n_b):
    n, d = node_feat.shape
    e = edge_feat.shape[0]
    nchunk = e // _CHUNK
    e_s = e // _NSLICE
    nchunk_s = nchunk // _NSLICE
    block_e = 5000
    nblk_s = e_s // block_e
    row = edge_index[0]
    col = edge_index[1]

    gidx = jnp.stack([row.reshape(nchunk, _CHUNK),
                      col.reshape(nchunk, _CHUNK) + n], axis=1)
    cidx = col.reshape(nchunk, 1, _CHUNK)

    b_e1r = b_e1.reshape(1, d)
    b_e2r = b_e2.reshape(1, d)
    g_er = ln_e_g.reshape(1, d)
    be_er = ln_e_b.reshape(1, d)

    table = _proj_nodes(node_feat, W_e1[:2 * d])
    out_edge = None
    ne_list = []
    for i in range(_NSLICE):
        eproj_i = _proj_edges(edge_feat, W_e1[2 * d:], b_e1r,
                              block_e=block_e, blk0=i * nblk_s, nblk=nblk_s)
        hpre_i = _sc_gather(table, eproj_i, gidx,
                            chunk0=i * nchunk_s, nchunk=nchunk_s)
        ne_i, out_edge = _edge_mlp2(out_edge, hpre_i, edge_feat, W_e2,
                                    b_e2r, g_er, be_er, block_e=block_e,
                                    blk0=i * nblk_s, nblk=nblk_s)
        ne_list.append(ne_i)
    parts = [_sc_scatter(ne_i, cidx, n, chunk0=i * nchunk_s,
                         nchunk=nchunk_s)
             for i, ne_i in enumerate(ne_list)]

    new_node = _node_mlp(node_feat, parts, W_n1[:d], W_n1[d:],
                         b_n1.reshape(1, d), W_n2, b_n2.reshape(1, d),
                         ln_n_g.reshape(1, d), ln_n_b.reshape(1, d),
                         block_n=1000)
    return (new_node, out_edge)
